# baseline (device time: 259172 ns/iter reference)
import functools

import jax
import jax.numpy as jnp
from jax import lax
from jax.experimental import pallas as pl
from jax.experimental.pallas import tpu as pltpu

N_DEV = 32
HEADS_PER = 8
DH = 128
SQ = 1024
D_MODEL = 1024
BLK = 64
SCALE = 0.08838834764831843
ROWS_PER_CHUNK = SQ // N_DEV


def _allreduce_body(p_ref, out_ref, comm_ref, send_sems, recv_sems):
    my = lax.axis_index("i")
    left = lax.rem(my - 1 + N_DEV, N_DEV)
    right = lax.rem(my + 1, N_DEV)

    barrier_sem = pltpu.get_barrier_semaphore()
    for nbr in (left, right):
        pl.semaphore_signal(
            barrier_sem, inc=1,
            device_id=(nbr,), device_id_type=pl.DeviceIdType.MESH,
        )
    pl.semaphore_wait(barrier_sem, 2)

    out_ref[...] = p_ref[...]
    rows = ROWS_PER_CHUNK

    def hop(h, c_send, c_recv, accumulate):
        send_slot = h % 2
        recv_slot = (h + 1) % 2
        comm_ref[send_slot, :, :] = out_ref[pl.ds(c_send * rows, rows), :]
        rdma = pltpu.make_async_remote_copy(
            src_ref=comm_ref.at[send_slot],
            dst_ref=comm_ref.at[recv_slot],
            send_sem=send_sems.at[send_slot],
            recv_sem=recv_sems.at[recv_slot],
            device_id=(right,),
            device_id_type=pl.DeviceIdType.MESH,
        )
        rdma.start()
        rdma.wait()
        if accumulate:
            cur = out_ref[pl.ds(c_recv * rows, rows), :]
            out_ref[pl.ds(c_recv * rows, rows), :] = cur + comm_ref[recv_slot, :, :]
        else:
            out_ref[pl.ds(c_recv * rows, rows), :] = comm_ref[recv_slot, :, :]

    for h in range(N_DEV - 1):
        c_send = lax.rem(my - h + 2 * N_DEV, N_DEV)
        c_recv = lax.rem(my - h - 1 + 2 * N_DEV, N_DEV)
        hop(h, c_send, c_recv, accumulate=True)

    for g in range(N_DEV - 1):
        h = N_DEV - 1 + g
        c_send = lax.rem(my + 1 - g + 2 * N_DEV, N_DEV)
        c_recv = lax.rem(my - g + 2 * N_DEV, N_DEV)
        hop(h, c_send, c_recv, accumulate=False)

    @functools.partial(pl.run_scoped, second_barrier=pltpu.SemaphoreType.REGULAR)
    def _(second_barrier):
        for nbr in (left, right):
            pl.semaphore_signal(
                second_barrier, inc=1,
                device_id=(nbr,), device_id_type=pl.DeviceIdType.MESH,
            )
        pl.semaphore_wait(second_barrier, 2)


def _ring_allreduce(partial):
    return pl.pallas_call(
        _allreduce_body,
        out_shape=jax.ShapeDtypeStruct((SQ, D_MODEL), jnp.float32),
        in_specs=[pl.BlockSpec(memory_space=pltpu.VMEM)],
        out_specs=pl.BlockSpec(memory_space=pltpu.VMEM),
        scratch_shapes=[
            pltpu.VMEM((2, ROWS_PER_CHUNK, D_MODEL), jnp.float32),
            pltpu.SemaphoreType.DMA((2,)),
            pltpu.SemaphoreType.DMA((2,)),
        ],
        compiler_params=pltpu.CompilerParams(collective_id=0),
    )(partial)


def kernel(x, Wq, K_ext, V_ext, Wo):
    my = lax.axis_index("i")

    xb = x[0].astype(jnp.bfloat16)
    Wqb = Wq.astype(jnp.bfloat16)
    Q = (xb @ Wqb).reshape(SQ, HEADS_PER, DH)

    K = lax.dynamic_slice_in_dim(K_ext[0], my * HEADS_PER, HEADS_PER, axis=1)
    V = lax.dynamic_slice_in_dim(V_ext[0], my * HEADS_PER, HEADS_PER, axis=1)
    K = K.astype(jnp.bfloat16)
    V = V.astype(jnp.bfloat16)

    scores = jnp.einsum(
        "ihd,jhd->hij", Q, K, preferred_element_type=jnp.float32
    ) * SCALE

    qb = (jnp.arange(SQ) // BLK)[:, None]
    kb = (jnp.arange(SQ) // BLK)[None, :]
    mask = kb <= qb
    scores = jnp.where(mask[None, :, :], scores, -1e9)
    w = jax.nn.softmax(scores, axis=-1)

    ctx = jnp.einsum(
        "hij,jhd->ihd", w.astype(jnp.bfloat16), V,
        preferred_element_type=jnp.float32,
    ).reshape(SQ, HEADS_PER * DH)

    partial = jnp.dot(
        ctx.astype(jnp.bfloat16), Wo.astype(jnp.bfloat16),
        preferred_element_type=jnp.float32,
    )

    out = _ring_allreduce(partial)
    return out[None, :, :]


# device time: 167283 ns/iter; 1.5493x vs baseline; 1.5493x over previous
import jax
import jax.numpy as jnp
from jax import lax
from jax.experimental import pallas as pl
from jax.experimental.pallas import tpu as pltpu

N_DEV = 32
HEADS_PER = 8
DH = 128
SQ = 1024
D_MODEL = 1024
BLK = 64
SCALE = 0.08838834764831843

N_STEPS = 5
HALF = [512, 256, 128, 64, 32]
OFFS = [0, 512, 768, 896, 960]
COMM_ROWS = 992


def _lid(cx, cy, cz):
    return 8 * cz + 2 * cy + jnp.bitwise_xor(cx, jnp.bitwise_and(cy, 1))


def _allreduce_body(p_ref, out_ref, comm_ref, rs_send, rs_recv, ag_send,
                    ag_recv):
    my = lax.axis_index("i")
    z = my // 8
    p = my % 8
    y = p // 2
    x = (p + y) % 2

    steps = [
        (_lid(1 - x, y, z), x),
        (_lid(x, jnp.bitwise_xor(y, 1), z), jnp.bitwise_and(y, 1)),
        (_lid(x, y, jnp.bitwise_xor(z, 1)), jnp.bitwise_and(z, 1)),
        (_lid(x, jnp.bitwise_xor(y, 2), z), y // 2),
        (_lid(x, y, jnp.bitwise_xor(z, 2)), z // 2),
    ]

    barrier_sem = pltpu.get_barrier_semaphore()
    for partner, _ in steps:
        pl.semaphore_signal(
            barrier_sem, inc=1,
            device_id=(partner,), device_id_type=pl.DeviceIdType.MESH,
        )
    pl.semaphore_wait(barrier_sem, N_STEPS)

    out_ref[...] = p_ref[...]

    seg_start = 0
    for k in range(N_STEPS):
        partner, b = steps[k]
        half = HALF[k]
        send_start = seg_start + (1 - b) * half
        keep_start = seg_start + b * half
        rdma = pltpu.make_async_remote_copy(
            src_ref=out_ref.at[pl.ds(send_start, half)],
            dst_ref=comm_ref.at[pl.ds(OFFS[k], half)],
            send_sem=rs_send.at[k],
            recv_sem=rs_recv.at[k],
            device_id=(partner,),
            device_id_type=pl.DeviceIdType.MESH,
        )
        rdma.start()
        rdma.wait()
        acc = out_ref[pl.ds(keep_start, half), :] + comm_ref[
            pl.ds(OFFS[k], half), :]
        out_ref[pl.ds(keep_start, half), :] = acc
        seg_start = keep_start

    for k in reversed(range(N_STEPS)):
        partner, b = steps[k]
        size = HALF[k]
        rdma = pltpu.make_async_remote_copy(
            src_ref=out_ref.at[pl.ds(seg_start, size)],
            dst_ref=out_ref.at[pl.ds(seg_start, size)],
            send_sem=ag_send.at[k],
            recv_sem=ag_recv.at[k],
            device_id=(partner,),
            device_id_type=pl.DeviceIdType.MESH,
        )
        rdma.start()
        rdma.wait()
        seg_start = seg_start - b * size


def _butterfly_allreduce(partial):
    return pl.pallas_call(
        _allreduce_body,
        out_shape=jax.ShapeDtypeStruct((SQ, D_MODEL), jnp.float32),
        in_specs=[pl.BlockSpec(memory_space=pltpu.VMEM)],
        out_specs=pl.BlockSpec(memory_space=pltpu.VMEM),
        scratch_shapes=[
            pltpu.VMEM((COMM_ROWS, D_MODEL), jnp.float32),
            pltpu.SemaphoreType.DMA((N_STEPS,)),
            pltpu.SemaphoreType.DMA((N_STEPS,)),
            pltpu.SemaphoreType.DMA((N_STEPS,)),
            pltpu.SemaphoreType.DMA((N_STEPS,)),
        ],
        compiler_params=pltpu.CompilerParams(collective_id=0),
    )(partial)


def kernel(x, Wq, K_ext, V_ext, Wo):
    my = lax.axis_index("i")

    xb = x[0].astype(jnp.bfloat16)
    Wqb = Wq.astype(jnp.bfloat16)
    Q = (xb @ Wqb).reshape(SQ, HEADS_PER, DH)

    K = lax.dynamic_slice_in_dim(K_ext[0], my * HEADS_PER, HEADS_PER, axis=1)
    V = lax.dynamic_slice_in_dim(V_ext[0], my * HEADS_PER, HEADS_PER, axis=1)
    K = K.astype(jnp.bfloat16)
    V = V.astype(jnp.bfloat16)

    scores = jnp.einsum(
        "ihd,jhd->hij", Q, K, preferred_element_type=jnp.float32
    ) * SCALE

    qb = (jnp.arange(SQ) // BLK)[:, None]
    kb = (jnp.arange(SQ) // BLK)[None, :]
    mask = kb <= qb
    scores = jnp.where(mask[None, :, :], scores, -1e9)
    w = jax.nn.softmax(scores, axis=-1)

    ctx = jnp.einsum(
        "hij,jhd->ihd", w.astype(jnp.bfloat16), V,
        preferred_element_type=jnp.float32,
    ).reshape(SQ, HEADS_PER * DH)

    partial = jnp.dot(
        ctx.astype(jnp.bfloat16), Wo.astype(jnp.bfloat16),
        preferred_element_type=jnp.float32,
    )

    out = _butterfly_allreduce(partial)
    return out[None, :, :]


# device time: 120970 ns/iter; 2.1424x vs baseline; 1.3828x over previous
import jax
import jax.numpy as jnp
from jax import lax
from jax.experimental import pallas as pl
from jax.experimental.pallas import tpu as pltpu

N_DEV = 32
HEADS_PER = 8
DH = 128
SQ = 1024
D_MODEL = 1024
BLK = 64
SCALE = 0.08838834764831843

N_STEPS = 5
HALF = [512, 256, 128, 64, 32]
OFFS = [0, 512, 768, 896, 960]
COMM_ROWS = 992


def _lid(cx, cy, cz):
    return 8 * cz + 2 * cy + jnp.bitwise_xor(cx, jnp.bitwise_and(cy, 1))


def _allreduce_body(p_ref, out_ref, comm_ref, agcomm_ref, stage_ref, rs_send,
                    rs_recv, ag_send, ag_recv):
    my = lax.axis_index("i")
    z = my // 8
    p = my % 8
    y = p // 2
    x = (p + y) % 2

    steps = [
        (_lid(1 - x, y, z), x),
        (_lid(x, jnp.bitwise_xor(y, 1), z), jnp.bitwise_and(y, 1)),
        (_lid(x, y, jnp.bitwise_xor(z, 1)), jnp.bitwise_and(z, 1)),
        (_lid(x, jnp.bitwise_xor(y, 2), z), y // 2),
        (_lid(x, y, jnp.bitwise_xor(z, 2)), z // 2),
    ]

    barrier_sem = pltpu.get_barrier_semaphore()
    for partner, _ in steps:
        pl.semaphore_signal(
            barrier_sem, inc=1,
            device_id=(partner,), device_id_type=pl.DeviceIdType.MESH,
        )
    pl.semaphore_wait(barrier_sem, N_STEPS)

    out_ref[...] = p_ref[...]

    seg_start = 0
    for k in range(N_STEPS):
        partner, b = steps[k]
        half = HALF[k]
        send_start = seg_start + (1 - b) * half
        keep_start = seg_start + b * half
        stage_ref[pl.ds(0, half), :] = out_ref[
            pl.ds(send_start, half), :].astype(jnp.bfloat16)
        rdma = pltpu.make_async_remote_copy(
            src_ref=stage_ref.at[pl.ds(0, half)],
            dst_ref=comm_ref.at[pl.ds(OFFS[k], half)],
            send_sem=rs_send.at[k],
            recv_sem=rs_recv.at[k],
            device_id=(partner,),
            device_id_type=pl.DeviceIdType.MESH,
        )
        rdma.start()
        rdma.wait()
        acc = out_ref[pl.ds(keep_start, half), :] + comm_ref[
            pl.ds(OFFS[k], half), :].astype(jnp.float32)
        out_ref[pl.ds(keep_start, half), :] = acc
        seg_start = keep_start

    for k in reversed(range(N_STEPS)):
        partner, b = steps[k]
        size = HALF[k]
        stage_ref[pl.ds(0, size), :] = out_ref[
            pl.ds(seg_start, size), :].astype(jnp.bfloat16)
        rdma = pltpu.make_async_remote_copy(
            src_ref=stage_ref.at[pl.ds(0, size)],
            dst_ref=agcomm_ref.at[pl.ds(OFFS[k], size)],
            send_sem=ag_send.at[k],
            recv_sem=ag_recv.at[k],
            device_id=(partner,),
            device_id_type=pl.DeviceIdType.MESH,
        )
        rdma.start()
        rdma.wait()
        partner_start = seg_start + (1 - 2 * b) * size
        out_ref[pl.ds(partner_start, size), :] = agcomm_ref[
            pl.ds(OFFS[k], size), :].astype(jnp.float32)
        seg_start = seg_start - b * size


def _butterfly_allreduce(partial):
    return pl.pallas_call(
        _allreduce_body,
        out_shape=jax.ShapeDtypeStruct((SQ, D_MODEL), jnp.float32),
        in_specs=[pl.BlockSpec(memory_space=pltpu.VMEM)],
        out_specs=pl.BlockSpec(memory_space=pltpu.VMEM),
        scratch_shapes=[
            pltpu.VMEM((COMM_ROWS, D_MODEL), jnp.bfloat16),
            pltpu.VMEM((COMM_ROWS, D_MODEL), jnp.bfloat16),
            pltpu.VMEM((HALF[0], D_MODEL), jnp.bfloat16),
            pltpu.SemaphoreType.DMA((N_STEPS,)),
            pltpu.SemaphoreType.DMA((N_STEPS,)),
            pltpu.SemaphoreType.DMA((N_STEPS,)),
            pltpu.SemaphoreType.DMA((N_STEPS,)),
        ],
        compiler_params=pltpu.CompilerParams(collective_id=0),
    )(partial)


def kernel(x, Wq, K_ext, V_ext, Wo):
    my = lax.axis_index("i")

    xb = x[0].astype(jnp.bfloat16)
    Wqb = Wq.astype(jnp.bfloat16)
    Q = (xb @ Wqb).reshape(SQ, HEADS_PER, DH)

    K = lax.dynamic_slice_in_dim(K_ext[0], my * HEADS_PER, HEADS_PER, axis=1)
    V = lax.dynamic_slice_in_dim(V_ext[0], my * HEADS_PER, HEADS_PER, axis=1)
    K = K.astype(jnp.bfloat16)
    V = V.astype(jnp.bfloat16)

    scores = jnp.einsum(
        "ihd,jhd->hij", Q, K, preferred_element_type=jnp.float32
    ) * SCALE

    qb = (jnp.arange(SQ) // BLK)[:, None]
    kb = (jnp.arange(SQ) // BLK)[None, :]
    mask = kb <= qb
    scores = jnp.where(mask[None, :, :], scores, -1e9)
    w = jax.nn.softmax(scores, axis=-1)

    ctx = jnp.einsum(
        "hij,jhd->ihd", w.astype(jnp.bfloat16), V,
        preferred_element_type=jnp.float32,
    ).reshape(SQ, HEADS_PER * DH)

    partial = jnp.dot(
        ctx.astype(jnp.bfloat16), Wo.astype(jnp.bfloat16),
        preferred_element_type=jnp.float32,
    )

    out = _butterfly_allreduce(partial)
    return out[None, :, :]
